# initial kernel scaffold (unmeasured)
import jax
import jax.numpy as jnp
from jax import lax
from jax.experimental import pallas as pl
from jax.experimental.pallas import tpu as pltpu


def kernel(
    x,
):
    def body(*refs):
        pass

    out_shape = jax.ShapeDtypeStruct(..., jnp.float32)
    return pl.pallas_call(body, out_shape=out_shape)(...)


# baseline (device time: 17792 ns/iter reference)
import jax
import jax.numpy as jnp
from jax import lax
from jax.experimental import pallas as pl
from jax.experimental.pallas import tpu as pltpu

HALF = 512


def kernel(x):
    _, m, n = x.shape

    def body(x_ref, out_ref, comm_ref, send_sem, recv_sem):
        my_x = lax.axis_index("x")
        my_y = lax.axis_index("y")
        my_z = lax.axis_index("z")
        partner = (my_x, 1 - my_y, my_z)

        barrier_sem = pltpu.get_barrier_semaphore()
        pl.semaphore_signal(
            barrier_sem, inc=1, device_id=partner,
            device_id_type=pl.DeviceIdType.MESH,
        )
        pl.semaphore_wait(barrier_sem, 1)

        rdma = pltpu.make_async_remote_copy(
            src_ref=x_ref.at[0, :, pl.ds((1 - my_y) * HALF, HALF)],
            dst_ref=comm_ref,
            send_sem=send_sem,
            recv_sem=recv_sem,
            device_id=partner,
            device_id_type=pl.DeviceIdType.MESH,
        )
        rdma.start()
        rdma.wait()

        out_ref[:, :] = comm_ref[:, :] + x_ref[0, :, pl.ds(my_y * HALF, HALF)]

    return pl.pallas_call(
        body,
        out_shape=jax.ShapeDtypeStruct((m, HALF), x.dtype),
        in_specs=[pl.BlockSpec(memory_space=pltpu.VMEM)],
        out_specs=pl.BlockSpec(memory_space=pltpu.VMEM),
        scratch_shapes=[
            pltpu.VMEM((m, HALF), x.dtype),
            pltpu.SemaphoreType.DMA,
            pltpu.SemaphoreType.DMA,
        ],
        compiler_params=pltpu.CompilerParams(collective_id=0),
    )(x)


# device time: 15513 ns/iter; 1.1469x vs baseline; 1.1469x over previous
import jax
import jax.numpy as jnp
from jax import lax
from jax.experimental import pallas as pl
from jax.experimental.pallas import tpu as pltpu

HALF = 512
ROWS = 256
K = 8
CH = ROWS // K


def kernel(x):
    _, m, n = x.shape

    def body(x_ref, out_ref, a_ref, b_ref, asend, arecv, bsend, brecv):
        my_x = lax.axis_index("x")
        my_y = lax.axis_index("y")
        my_z = lax.axis_index("z")
        partner = (my_x, 1 - my_y, my_z)
        sibling = (my_x, my_y, 1 - my_z)

        barrier_sem = pltpu.get_barrier_semaphore()
        for nbr in (partner, sibling):
            pl.semaphore_signal(
                barrier_sem, inc=1, device_id=nbr,
                device_id_type=pl.DeviceIdType.MESH,
            )
        pl.semaphore_wait(barrier_sem, 2)

        row0 = my_z * ROWS
        orow0 = (1 - my_z) * ROWS
        col_mine = my_y * HALF
        col_partner = (1 - my_y) * HALF

        a = []
        for c in range(K):
            rd = pltpu.make_async_remote_copy(
                src_ref=x_ref.at[
                    0, pl.ds(row0 + c * CH, CH), pl.ds(col_partner, HALF)
                ],
                dst_ref=a_ref.at[pl.ds(c * CH, CH), :],
                send_sem=asend.at[c],
                recv_sem=arecv.at[c],
                device_id=partner,
                device_id_type=pl.DeviceIdType.MESH,
            )
            rd.start()
            a.append(rd)

        b = []
        for c in range(K):
            a[c].wait_recv()
            rd = pltpu.make_async_remote_copy(
                src_ref=a_ref.at[pl.ds(c * CH, CH), :],
                dst_ref=b_ref.at[pl.ds(c * CH, CH), :],
                send_sem=bsend.at[c],
                recv_sem=brecv.at[c],
                device_id=sibling,
                device_id_type=pl.DeviceIdType.MESH,
            )
            rd.start()
            b.append(rd)
            out_ref[pl.ds(row0 + c * CH, CH), :] = (
                a_ref[pl.ds(c * CH, CH), :]
                + x_ref[0, pl.ds(row0 + c * CH, CH), pl.ds(col_mine, HALF)]
            )

        for c in range(K):
            b[c].wait_recv()
            out_ref[pl.ds(orow0 + c * CH, CH), :] = (
                b_ref[pl.ds(c * CH, CH), :]
                + x_ref[0, pl.ds(orow0 + c * CH, CH), pl.ds(col_mine, HALF)]
            )

        for c in range(K):
            a[c].wait_send()
            b[c].wait_send()

    return pl.pallas_call(
        body,
        out_shape=jax.ShapeDtypeStruct((m, HALF), x.dtype),
        in_specs=[pl.BlockSpec(memory_space=pltpu.VMEM)],
        out_specs=pl.BlockSpec(memory_space=pltpu.VMEM),
        scratch_shapes=[
            pltpu.VMEM((ROWS, HALF), x.dtype),
            pltpu.VMEM((ROWS, HALF), x.dtype),
            pltpu.SemaphoreType.DMA((K,)),
            pltpu.SemaphoreType.DMA((K,)),
            pltpu.SemaphoreType.DMA((K,)),
            pltpu.SemaphoreType.DMA((K,)),
        ],
        compiler_params=pltpu.CompilerParams(collective_id=0),
    )(x)
